# baseline (speedup 1.0000x reference)
import dataclasses
import functools

import jax
import jax.numpy as jnp
from jax import lax
from jax.experimental import pallas as pl
from jax.experimental.pallas import tpu as pltpu
from jax.experimental.pallas import tpu_sc as plsc

D = 128
NC = 2
NS = 16
NW = NC * NS


def _prep_body(s1, s2, p1, p2, x, w, tsrc, tdst):
    xb = x[...].astype(jnp.bfloat16)
    wb = w[...].astype(jnp.bfloat16)
    zero = jnp.zeros_like(xb)
    tsrc[:, 0:D] = (s1[...] + s2[...]).astype(jnp.bfloat16)
    tsrc[:, D:2 * D] = xb
    tsrc[:, 2 * D:3 * D] = wb
    tsrc[:, 3 * D:4 * D] = zero
    tdst[:, 0:D] = (p1[...] + p2[...]).astype(jnp.bfloat16)
    tdst[:, D:2 * D] = xb
    tdst[:, 2 * D:3 * D] = wb
    tdst[:, 3 * D:4 * D] = zero


def _build_tables(s1, s2, p1, p2, x, w):
    n = s1.shape[0]
    blk = 2000
    grid = (n // blk,)
    in_spec = pl.BlockSpec((blk, D), lambda i: (i, 0))
    out_spec = pl.BlockSpec((blk, 4 * D), lambda i: (i, 0))
    return pl.pallas_call(
        _prep_body,
        grid=grid,
        in_specs=[in_spec] * 6,
        out_specs=[out_spec, out_spec],
        out_shape=[jax.ShapeDtypeStruct((n, 4 * D), jnp.bfloat16)] * 2,
    )(s1, s2, p1, p2, x, w)


def _sc_gather_combine(tsrc32, tdst32, src_idx, dst_idx):
    e = src_idx.shape[0]
    dtin = tsrc32.shape[1]
    dt = 192
    ew = e // NW
    wnd = 80
    nch = ew // wnd
    mesh = plsc.VectorSubcoreMesh(core_axis_name="c", subcore_axis_name="s")
    cp = pltpu.CompilerParams()
    if "needs_layout_passes" in pltpu.CompilerParams.__dataclass_fields__:
        cp = dataclasses.replace(cp, needs_layout_passes=False)

    @functools.partial(
        pl.kernel,
        out_type=jax.ShapeDtypeStruct((e, dt), jnp.int32),
        mesh=mesh,
        compiler_params=cp,
        scratch_types=[
            pltpu.VMEM((ew,), jnp.int32),
            pltpu.VMEM((ew,), jnp.int32),
            pltpu.VMEM((wnd, dtin), jnp.int32),
            pltpu.VMEM((wnd, dtin), jnp.int32),
            pltpu.VMEM((wnd, dt), jnp.int32),
            pltpu.SemaphoreType.DMA,
            pltpu.SemaphoreType.DMA,
        ],
    )
    def k(tsrc_hbm, tdst_hbm, si_hbm, di_hbm, u_hbm,
          si_v, di_v, a_v, b_v, u_v, sem_a, sem_b):
        wid = lax.axis_index("s") * NC + lax.axis_index("c")
        base = wid * ew
        pltpu.sync_copy(si_hbm.at[pl.ds(base, ew)], si_v)
        pltpu.sync_copy(di_hbm.at[pl.ds(base, ew)], di_v)

        @pl.loop(0, nch)
        def _chunk(i):
            off = i * wnd
            ca = pltpu.async_copy(tsrc_hbm.at[si_v.at[pl.ds(off, wnd)]],
                                  a_v, sem_a)
            cb = pltpu.async_copy(tdst_hbm.at[di_v.at[pl.ds(off, wnd)]],
                                  b_v, sem_b)
            ca.wait()
            cb.wait()

            @pl.loop(0, wnd)
            def _row(r):
                for j in range(dt // 16):
                    slc = (r, pl.ds(j * 16, 16))
                    av = plsc.bitcast(a_v.at[*slc][...], jnp.bfloat16)
                    bv = plsc.bitcast(b_v.at[*slc][...], jnp.bfloat16)
                    res = av + bv if j < 4 else av * bv
                    u_v.at[*slc][...] = plsc.bitcast(res, jnp.int32)

            pltpu.sync_copy(u_v, u_hbm.at[pl.ds(base + off, wnd)])

    return k(tsrc32, tdst32, src_idx, dst_idx)


def _leaky(v):
    return jnp.maximum(v, 0.01 * v)


def _mlp_body(u, w1, b1, wl, wh1, wh2, cst, o):
    ub = u[...]
    z = jnp.dot(_leaky(ub[:, 0:D]), w1[...],
                preferred_element_type=jnp.float32) + b1[...]
    hl = _leaky(z).astype(jnp.bfloat16)
    o[...] = (jnp.dot(hl, wl[...], preferred_element_type=jnp.float32)
              + jnp.dot(ub[:, D:2 * D], wh1[...],
                        preferred_element_type=jnp.float32)
              + jnp.dot(ub[:, 2 * D:3 * D], wh2[...],
                        preferred_element_type=jnp.float32)
              + cst[...])


def _tc_mlp(u, w1, b1, wl, wh1, wh2, cst):
    e = u.shape[0]
    blk = 2000
    grid = (e // blk,)
    full = lambda i: (0, 0)
    return pl.pallas_call(
        _mlp_body,
        grid=grid,
        in_specs=[
            pl.BlockSpec((blk, 3 * D), lambda i: (i, 0)),
            pl.BlockSpec((D, D), full),
            pl.BlockSpec((1, D), full),
            pl.BlockSpec((D, 1), full),
            pl.BlockSpec((D, 1), full),
            pl.BlockSpec((D, 1), full),
            pl.BlockSpec((1, 1), full),
        ],
        out_specs=pl.BlockSpec((blk, 1), lambda i: (i, 0)),
        out_shape=jax.ShapeDtypeStruct((e, 1), jnp.float32),
    )(u, w1, b1, wl, wh1, wh2, cst)


def kernel(s1, s2, p1, p2, x, w, edge_index, W1, b1, WL, bL, Wh1, bh1, Wh2, bh2):
    n = s1.shape[0]
    e = edge_index.shape[1]
    tsrc, tdst = _build_tables(s1, s2, p1, p2, x, w)
    tsrc32 = jax.lax.bitcast_convert_type(
        tsrc.reshape(n, 2 * D, 2), jnp.int32)
    tdst32 = jax.lax.bitcast_convert_type(
        tdst.reshape(n, 2 * D, 2), jnp.int32)
    src = edge_index[0].astype(jnp.int32)
    dst = edge_index[1].astype(jnp.int32)
    u32 = _sc_gather_combine(tsrc32, tdst32, src, dst)
    u = jax.lax.bitcast_convert_type(u32, jnp.bfloat16).reshape(e, 3 * D)
    cst = (bL + bh1 + bh2).reshape(1, 1).astype(jnp.float32)
    bf = jnp.bfloat16
    return _tc_mlp(u, W1.astype(bf), b1.reshape(1, D), WL.astype(bf),
                   Wh1.astype(bf), Wh2.astype(bf), cst)


# speedup vs baseline: 3.5869x; 3.5869x over previous
import dataclasses
import functools

import jax
import jax.numpy as jnp
from jax import lax
from jax.experimental import pallas as pl
from jax.experimental.pallas import tpu as pltpu
from jax.experimental.pallas import tpu_sc as plsc

D = 128
H = D // 2
NC = 2
NS = 16
NW = NC * NS


def _pack(feat):
    hi = jax.lax.bitcast_convert_type(feat[:, 0:H], jnp.uint16)
    lo = jax.lax.bitcast_convert_type(feat[:, H:2 * H], jnp.uint16)
    word = (hi.astype(jnp.uint32) << 16) | lo.astype(jnp.uint32)
    return jax.lax.bitcast_convert_type(word, jnp.int32)


def _prep_body(s1, s2, p1, p2, x, w, tsrc, tdst):
    xb = _pack(x[...].astype(jnp.bfloat16))
    wb = _pack(w[...].astype(jnp.bfloat16))
    zero = jnp.zeros_like(xb)
    tsrc[:, 0:H] = _pack((s1[...] + s2[...]).astype(jnp.bfloat16))
    tsrc[:, H:2 * H] = xb
    tsrc[:, 2 * H:3 * H] = wb
    tsrc[:, 3 * H:4 * H] = zero
    tdst[:, 0:H] = _pack((p1[...] + p2[...]).astype(jnp.bfloat16))
    tdst[:, H:2 * H] = xb
    tdst[:, 2 * H:3 * H] = wb
    tdst[:, 3 * H:4 * H] = zero


def _build_tables(s1, s2, p1, p2, x, w):
    n = s1.shape[0]
    blk = 2000
    grid = (n // blk,)
    in_spec = pl.BlockSpec((blk, D), lambda i: (i, 0))
    out_spec = pl.BlockSpec((blk, 4 * H), lambda i: (i, 0))
    return pl.pallas_call(
        _prep_body,
        grid=grid,
        in_specs=[in_spec] * 6,
        out_specs=[out_spec, out_spec],
        out_shape=[jax.ShapeDtypeStruct((n, 4 * H), jnp.int32)] * 2,
    )(s1, s2, p1, p2, x, w)


def _sc_gather_combine(tsrc32, tdst32, src_idx, dst_idx):
    e = src_idx.shape[0]
    dtin = tsrc32.shape[1]
    dt = 3 * H
    ew = e // NW
    wnd = 80
    nch = ew // wnd
    mesh = plsc.VectorSubcoreMesh(core_axis_name="c", subcore_axis_name="s")
    cp = pltpu.CompilerParams()
    if "needs_layout_passes" in pltpu.CompilerParams.__dataclass_fields__:
        cp = dataclasses.replace(cp, needs_layout_passes=False)

    @functools.partial(
        pl.kernel,
        out_type=jax.ShapeDtypeStruct((e, dt), jnp.int32),
        mesh=mesh,
        compiler_params=cp,
        scratch_types=[
            pltpu.VMEM((ew,), jnp.int32),
            pltpu.VMEM((ew,), jnp.int32),
            pltpu.VMEM((wnd, dtin), jnp.int32),
            pltpu.VMEM((wnd, dtin), jnp.int32),
            pltpu.VMEM((wnd, dt), jnp.int32),
            pltpu.SemaphoreType.DMA,
            pltpu.SemaphoreType.DMA,
        ],
    )
    def k(tsrc_hbm, tdst_hbm, si_hbm, di_hbm, u_hbm,
          si_v, di_v, a_v, b_v, u_v, sem_a, sem_b):
        wid = lax.axis_index("s") * NC + lax.axis_index("c")
        base = wid * ew
        pltpu.sync_copy(si_hbm.at[pl.ds(base, ew)], si_v)
        pltpu.sync_copy(di_hbm.at[pl.ds(base, ew)], di_v)

        @pl.loop(0, nch)
        def _chunk(i):
            off = i * wnd
            ca = pltpu.async_copy(tsrc_hbm.at[si_v.at[pl.ds(off, wnd)]],
                                  a_v, sem_a)
            cb = pltpu.async_copy(tdst_hbm.at[di_v.at[pl.ds(off, wnd)]],
                                  b_v, sem_b)
            ca.wait()
            cb.wait()

            @pl.loop(0, wnd)
            def _row(r):
                for j in range(dt // 16):
                    slc = (r, pl.ds(j * 16, 16))
                    av = plsc.bitcast(a_v.at[*slc][...], jnp.bfloat16)
                    bv = plsc.bitcast(b_v.at[*slc][...], jnp.bfloat16)
                    res = av + bv if j < 4 else av * bv
                    u_v.at[*slc][...] = plsc.bitcast(res, jnp.int32)

            pltpu.sync_copy(u_v, u_hbm.at[pl.ds(base + off, wnd)])

    return k(tsrc32, tdst32, src_idx, dst_idx)


def _leaky(v):
    return jnp.maximum(v, 0.01 * v)


def _unpack(sec):
    hi = jax.lax.bitcast_convert_type(sec & jnp.uint32(0xFFFF0000),
                                      jnp.float32)
    lo = jax.lax.bitcast_convert_type(sec << 16, jnp.float32)
    return jnp.concatenate([hi, lo], axis=1)


def _mlp_body(u, w1, b1, wl, wh1, wh2, cst, o):
    uu = jax.lax.bitcast_convert_type(u[...], jnp.uint32)
    c = _unpack(uu[:, 0:H])
    xp = _unpack(uu[:, H:2 * H])
    wp = _unpack(uu[:, 2 * H:3 * H])
    z = jnp.dot(_leaky(c).astype(jnp.bfloat16), w1[...],
                preferred_element_type=jnp.float32) + b1[...]
    hl = _leaky(z).astype(jnp.bfloat16)
    o[...] = (jnp.dot(hl, wl[...], preferred_element_type=jnp.float32)
              + jnp.dot(xp.astype(jnp.bfloat16), wh1[...],
                        preferred_element_type=jnp.float32)
              + jnp.dot(wp.astype(jnp.bfloat16), wh2[...],
                        preferred_element_type=jnp.float32)
              + cst[...])


def _tc_mlp(u, w1, b1, wl, wh1, wh2, cst):
    e = u.shape[0]
    blk = 2000
    grid = (e // blk,)
    full = lambda i: (0, 0)
    return pl.pallas_call(
        _mlp_body,
        grid=grid,
        in_specs=[
            pl.BlockSpec((blk, 3 * H), lambda i: (i, 0)),
            pl.BlockSpec((D, D), full),
            pl.BlockSpec((1, D), full),
            pl.BlockSpec((D, 1), full),
            pl.BlockSpec((D, 1), full),
            pl.BlockSpec((D, 1), full),
            pl.BlockSpec((1, 1), full),
        ],
        out_specs=pl.BlockSpec((blk, 1), lambda i: (i, 0)),
        out_shape=jax.ShapeDtypeStruct((e, 1), jnp.float32),
    )(u, w1, b1, wl, wh1, wh2, cst)


def kernel(s1, s2, p1, p2, x, w, edge_index, W1, b1, WL, bL, Wh1, bh1, Wh2, bh2):
    tsrc32, tdst32 = _build_tables(s1, s2, p1, p2, x, w)
    src = edge_index[0].astype(jnp.int32)
    dst = edge_index[1].astype(jnp.int32)
    u32 = _sc_gather_combine(tsrc32, tdst32, src, dst)
    cst = (bL + bh1 + bh2).reshape(1, 1).astype(jnp.float32)
    bf = jnp.bfloat16
    return _tc_mlp(u32, W1.astype(bf), b1.reshape(1, D), WL.astype(bf),
                   Wh1.astype(bf), Wh2.astype(bf), cst)


# speedup vs baseline: 4.5192x; 1.2599x over previous
import dataclasses
import functools

import jax
import jax.numpy as jnp
from jax import lax
from jax.experimental import pallas as pl
from jax.experimental.pallas import tpu as pltpu
from jax.experimental.pallas import tpu_sc as plsc

D = 128
H = D // 2
NC = 2
NS = 16
NW = NC * NS


def _pack(feat):
    hi = jax.lax.bitcast_convert_type(feat[:, 0:H], jnp.uint16)
    lo = jax.lax.bitcast_convert_type(feat[:, H:2 * H], jnp.uint16)
    word = (hi.astype(jnp.uint32) << 16) | lo.astype(jnp.uint32)
    return jax.lax.bitcast_convert_type(word, jnp.int32)


def _prep_body(s1, s2, p1, p2, x, w, tsrc, tdst):
    xb = _pack(x[...].astype(jnp.bfloat16))
    wb = _pack(w[...].astype(jnp.bfloat16))
    zero = jnp.zeros_like(xb)
    tsrc[:, 0:H] = _pack((s1[...] + s2[...]).astype(jnp.bfloat16))
    tsrc[:, H:2 * H] = xb
    tsrc[:, 2 * H:3 * H] = wb
    tsrc[:, 3 * H:4 * H] = zero
    tdst[:, 0:H] = _pack((p1[...] + p2[...]).astype(jnp.bfloat16))
    tdst[:, H:2 * H] = xb
    tdst[:, 2 * H:3 * H] = wb
    tdst[:, 3 * H:4 * H] = zero


def _build_tables(s1, s2, p1, p2, x, w):
    n = s1.shape[0]
    blk = 2000
    grid = (n // blk,)
    in_spec = pl.BlockSpec((blk, D), lambda i: (i, 0))
    out_spec = pl.BlockSpec((blk, 4 * H), lambda i: (i, 0))
    return pl.pallas_call(
        _prep_body,
        grid=grid,
        in_specs=[in_spec] * 6,
        out_specs=[out_spec, out_spec],
        out_shape=[jax.ShapeDtypeStruct((n, 4 * H), jnp.int32)] * 2,
    )(s1, s2, p1, p2, x, w)


def _sc_gather_combine(tsrc32, tdst32, src_idx, dst_idx):
    e = src_idx.shape[0]
    dtin = tsrc32.shape[1]
    dt = 3 * H
    ew = e // NW
    wnd = 40
    nch = ew // wnd
    mesh = plsc.VectorSubcoreMesh(core_axis_name="c", subcore_axis_name="s")
    cp = pltpu.CompilerParams()
    if "needs_layout_passes" in pltpu.CompilerParams.__dataclass_fields__:
        cp = dataclasses.replace(cp, needs_layout_passes=False)

    @functools.partial(
        pl.kernel,
        out_type=jax.ShapeDtypeStruct((e, dt), jnp.int32),
        mesh=mesh,
        compiler_params=cp,
        scratch_types=[
            pltpu.VMEM((ew,), jnp.int32),
            pltpu.VMEM((ew,), jnp.int32),
            pltpu.VMEM((wnd, dtin), jnp.int32),
            pltpu.VMEM((wnd, dtin), jnp.int32),
            pltpu.VMEM((wnd, dtin), jnp.int32),
            pltpu.VMEM((wnd, dtin), jnp.int32),
            pltpu.VMEM((wnd, dt), jnp.int32),
            pltpu.VMEM((wnd, dt), jnp.int32),
            pltpu.SemaphoreType.DMA,
            pltpu.SemaphoreType.DMA,
            pltpu.SemaphoreType.DMA,
            pltpu.SemaphoreType.DMA,
            pltpu.SemaphoreType.DMA,
            pltpu.SemaphoreType.DMA,
        ],
    )
    def k(tsrc_hbm, tdst_hbm, si_hbm, di_hbm, u_hbm,
          si_v, di_v, a0_v, a1_v, b0_v, b1_v, u0_v, u1_v,
          sa0, sa1, sb0, sb1, su0, su1):
        wid = lax.axis_index("s") * NC + lax.axis_index("c")
        base = wid * ew
        pltpu.sync_copy(si_hbm.at[pl.ds(base, ew)], si_v)
        pltpu.sync_copy(di_hbm.at[pl.ds(base, ew)], di_v)

        a_bufs, b_bufs, u_bufs = (a0_v, a1_v), (b0_v, b1_v), (u0_v, u1_v)
        sa, sb, su = (sa0, sa1), (sb0, sb1), (su0, su1)

        def fire(g, pb):
            off = g * wnd
            pltpu.async_copy(tsrc_hbm.at[si_v.at[pl.ds(off, wnd)]],
                             a_bufs[pb], sa[pb])
            pltpu.async_copy(tdst_hbm.at[di_v.at[pl.ds(off, wnd)]],
                             b_bufs[pb], sb[pb])

        fire(0, 0)
        fire(1, 1)

        @pl.loop(0, nch, step=2)
        def _chunk(i):
            for pb in range(2):
                g = i + pb
                pltpu.make_async_copy(tsrc_hbm.at[pl.ds(0, wnd)],
                                      a_bufs[pb], sa[pb]).wait()
                pltpu.make_async_copy(tsrc_hbm.at[pl.ds(0, wnd)],
                                      b_bufs[pb], sb[pb]).wait()

                @pl.when(i >= 2)
                def _():
                    pltpu.make_async_copy(u_bufs[pb],
                                          u_hbm.at[pl.ds(base, wnd)],
                                          su[pb]).wait()

                a_v, b_v, u_v = a_bufs[pb], b_bufs[pb], u_bufs[pb]

                @pl.loop(0, wnd)
                def _row(r):
                    for j in range(dt // 16):
                        slc = (r, pl.ds(j * 16, 16))
                        av = plsc.bitcast(a_v.at[*slc][...], jnp.bfloat16)
                        bv = plsc.bitcast(b_v.at[*slc][...], jnp.bfloat16)
                        res = av + bv if j < 4 else av * bv
                        u_v.at[*slc][...] = plsc.bitcast(res, jnp.int32)

                pltpu.async_copy(u_v, u_hbm.at[pl.ds(base + g * wnd, wnd)],
                                 su[pb])

                @pl.when(g + 2 < nch)
                def _():
                    fire(g + 2, pb)

        for pb in range(2):
            pltpu.make_async_copy(u_bufs[pb], u_hbm.at[pl.ds(base, wnd)],
                                  su[pb]).wait()

    return k(tsrc32, tdst32, src_idx, dst_idx)


def _leaky(v):
    return jnp.maximum(v, 0.01 * v)


def _unpack(sec):
    hi = jax.lax.bitcast_convert_type(sec & jnp.uint32(0xFFFF0000),
                                      jnp.float32)
    lo = jax.lax.bitcast_convert_type(sec << 16, jnp.float32)
    return jnp.concatenate([hi, lo], axis=1)


def _mlp_body(u, w1, b1, wl, wh1, wh2, cst, o):
    uu = jax.lax.bitcast_convert_type(u[...], jnp.uint32)
    c = _unpack(uu[:, 0:H])
    xp = _unpack(uu[:, H:2 * H])
    wp = _unpack(uu[:, 2 * H:3 * H])
    z = jnp.dot(_leaky(c).astype(jnp.bfloat16), w1[...],
                preferred_element_type=jnp.float32) + b1[...]
    hl = _leaky(z).astype(jnp.bfloat16)
    o[...] = (jnp.dot(hl, wl[...], preferred_element_type=jnp.float32)
              + jnp.dot(xp.astype(jnp.bfloat16), wh1[...],
                        preferred_element_type=jnp.float32)
              + jnp.dot(wp.astype(jnp.bfloat16), wh2[...],
                        preferred_element_type=jnp.float32)
              + cst[...])


def _tc_mlp(u, w1, b1, wl, wh1, wh2, cst):
    e = u.shape[0]
    blk = 2000
    grid = (e // blk,)
    full = lambda i: (0, 0)
    return pl.pallas_call(
        _mlp_body,
        grid=grid,
        in_specs=[
            pl.BlockSpec((blk, 3 * H), lambda i: (i, 0)),
            pl.BlockSpec((D, D), full),
            pl.BlockSpec((1, D), full),
            pl.BlockSpec((D, 1), full),
            pl.BlockSpec((D, 1), full),
            pl.BlockSpec((D, 1), full),
            pl.BlockSpec((1, 1), full),
        ],
        out_specs=pl.BlockSpec((blk, 1), lambda i: (i, 0)),
        out_shape=jax.ShapeDtypeStruct((e, 1), jnp.float32),
    )(u, w1, b1, wl, wh1, wh2, cst)


def kernel(s1, s2, p1, p2, x, w, edge_index, W1, b1, WL, bL, Wh1, bh1, Wh2, bh2):
    tsrc32, tdst32 = _build_tables(s1, s2, p1, p2, x, w)
    src = edge_index[0].astype(jnp.int32)
    dst = edge_index[1].astype(jnp.int32)
    u32 = _sc_gather_combine(tsrc32, tdst32, src, dst)
    cst = (bL + bh1 + bh2).reshape(1, 1).astype(jnp.float32)
    bf = jnp.bfloat16
    return _tc_mlp(u32, W1.astype(bf), b1.reshape(1, D), WL.astype(bf),
                   Wh1.astype(bf), Wh2.astype(bf), cst)


# speedup vs baseline: 4.9989x; 1.1062x over previous
import dataclasses
import functools

import jax
import jax.numpy as jnp
from jax import lax
from jax.experimental import pallas as pl
from jax.experimental.pallas import tpu as pltpu
from jax.experimental.pallas import tpu_sc as plsc

D = 128
H = D // 2
NC = 2
NS = 16
NW = NC * NS


def _pack(feat):
    hi = jax.lax.bitcast_convert_type(feat[:, 0:H], jnp.uint16)
    lo = jax.lax.bitcast_convert_type(feat[:, H:2 * H], jnp.uint16)
    word = (hi.astype(jnp.uint32) << 16) | lo.astype(jnp.uint32)
    return jax.lax.bitcast_convert_type(word, jnp.int32)


def _prep_body(s1, s2, p1, p2, x, w, tsrc, tdst):
    xb = _pack(x[...].astype(jnp.bfloat16))
    wb = _pack(w[...].astype(jnp.bfloat16))
    zero = jnp.zeros_like(xb)
    tsrc[:, 0:H] = _pack((s1[...] + s2[...]).astype(jnp.bfloat16))
    tsrc[:, H:2 * H] = xb
    tsrc[:, 2 * H:3 * H] = wb
    tsrc[:, 3 * H:4 * H] = zero
    tdst[:, 0:H] = _pack((p1[...] + p2[...]).astype(jnp.bfloat16))
    tdst[:, H:2 * H] = xb
    tdst[:, 2 * H:3 * H] = wb
    tdst[:, 3 * H:4 * H] = zero


def _build_tables(s1, s2, p1, p2, x, w):
    n = s1.shape[0]
    blk = 2000
    grid = (n // blk,)
    in_spec = pl.BlockSpec((blk, D), lambda i: (i, 0))
    out_spec = pl.BlockSpec((blk, 4 * H), lambda i: (i, 0))
    return pl.pallas_call(
        _prep_body,
        grid=grid,
        in_specs=[in_spec] * 6,
        out_specs=[out_spec, out_spec],
        out_shape=[jax.ShapeDtypeStruct((n, 4 * H), jnp.int32)] * 2,
    )(s1, s2, p1, p2, x, w)


def _sc_gather_combine(tsrc32, tdst32, src_idx, dst_idx):
    e = src_idx.shape[0]
    dtin = tsrc32.shape[1]
    dt = 3 * H
    ew = e // NW
    wnd = 40
    nch = ew // wnd
    mesh = plsc.VectorSubcoreMesh(core_axis_name="c", subcore_axis_name="s")
    cp = pltpu.CompilerParams()
    if "needs_layout_passes" in pltpu.CompilerParams.__dataclass_fields__:
        cp = dataclasses.replace(cp, needs_layout_passes=False)

    @functools.partial(
        pl.kernel,
        out_type=jax.ShapeDtypeStruct((e, dt), jnp.int32),
        mesh=mesh,
        compiler_params=cp,
        scratch_types=[
            pltpu.VMEM((ew,), jnp.int32),
            pltpu.VMEM((ew,), jnp.int32),
            pltpu.VMEM((wnd, dtin), jnp.int32),
            pltpu.VMEM((wnd, dtin), jnp.int32),
            pltpu.VMEM((wnd, dtin), jnp.int32),
            pltpu.VMEM((wnd, dtin), jnp.int32),
            pltpu.VMEM((wnd, dt), jnp.int32),
            pltpu.VMEM((wnd, dt), jnp.int32),
            pltpu.SemaphoreType.DMA,
            pltpu.SemaphoreType.DMA,
            pltpu.SemaphoreType.DMA,
            pltpu.SemaphoreType.DMA,
            pltpu.SemaphoreType.DMA,
            pltpu.SemaphoreType.DMA,
        ],
    )
    def k(tsrc_hbm, tdst_hbm, si_hbm, di_hbm, u_hbm,
          si_v, di_v, a0_v, a1_v, b0_v, b1_v, u0_v, u1_v,
          sa0, sa1, sb0, sb1, su0, su1):
        wid = lax.axis_index("s") * NC + lax.axis_index("c")
        base = wid * ew
        pltpu.sync_copy(si_hbm.at[pl.ds(base, ew)], si_v)
        pltpu.sync_copy(di_hbm.at[pl.ds(base, ew)], di_v)

        a_bufs, b_bufs, u_bufs = (a0_v, a1_v), (b0_v, b1_v), (u0_v, u1_v)
        sa, sb, su = (sa0, sa1), (sb0, sb1), (su0, su1)

        def fire(g, pb):
            off = g * wnd
            pltpu.async_copy(tsrc_hbm.at[si_v.at[pl.ds(off, wnd)]],
                             a_bufs[pb], sa[pb])
            pltpu.async_copy(tdst_hbm.at[di_v.at[pl.ds(off, wnd)]],
                             b_bufs[pb], sb[pb])

        fire(0, 0)
        fire(1, 1)

        @pl.loop(0, nch, step=2)
        def _chunk(i):
            for pb in range(2):
                g = i + pb
                pltpu.make_async_copy(tsrc_hbm.at[pl.ds(0, wnd)],
                                      a_bufs[pb], sa[pb]).wait()
                pltpu.make_async_copy(tsrc_hbm.at[pl.ds(0, wnd)],
                                      b_bufs[pb], sb[pb]).wait()

                @pl.when(i >= 2)
                def _():
                    pltpu.make_async_copy(u_bufs[pb],
                                          u_hbm.at[pl.ds(base, wnd)],
                                          su[pb]).wait()

                a_v, b_v, u_v = a_bufs[pb], b_bufs[pb], u_bufs[pb]

                @pl.loop(0, wnd)
                def _row(r):
                    for j in range(dt // 16):
                        slc = (r, pl.ds(j * 16, 16))
                        av = plsc.bitcast(a_v.at[*slc][...], jnp.bfloat16)
                        bv = plsc.bitcast(b_v.at[*slc][...], jnp.bfloat16)
                        res = av + bv if j < 4 else av * bv
                        u_v.at[*slc][...] = plsc.bitcast(res, jnp.int32)

                pltpu.async_copy(u_v, u_hbm.at[pl.ds(base + g * wnd, wnd)],
                                 su[pb])

                @pl.when(g + 2 < nch)
                def _():
                    fire(g + 2, pb)

        for pb in range(2):
            pltpu.make_async_copy(u_bufs[pb], u_hbm.at[pl.ds(base, wnd)],
                                  su[pb]).wait()

    return k(tsrc32, tdst32, src_idx, dst_idx)


def _leaky(v):
    return jnp.maximum(v, 0.01 * v)


def _unpack(sec):
    hi = jax.lax.bitcast_convert_type(sec & jnp.uint32(0xFFFF0000),
                                      jnp.float32)
    lo = jax.lax.bitcast_convert_type(sec << 16, jnp.float32)
    return jnp.concatenate([hi, lo], axis=1)


def _mlp_body(u, w1, b1, wcat, cst, o):
    uu = jax.lax.bitcast_convert_type(u[...], jnp.uint32)
    c = _unpack(uu[:, 0:H])
    xp = _unpack(uu[:, H:2 * H])
    wp = _unpack(uu[:, 2 * H:3 * H])
    z = jnp.dot(_leaky(c).astype(jnp.bfloat16), w1[...],
                preferred_element_type=jnp.float32) + b1[...]
    a2 = jnp.concatenate([_leaky(z), xp, wp], axis=1).astype(jnp.bfloat16)
    o[...] = jnp.dot(a2, wcat[...],
                     preferred_element_type=jnp.float32) + cst[...]


def _tc_mlp(u, w1, b1, wcat, cst):
    e = u.shape[0]
    blk = 4000
    grid = (e // blk,)
    full = lambda i: (0, 0)
    return pl.pallas_call(
        _mlp_body,
        grid=grid,
        in_specs=[
            pl.BlockSpec((blk, 3 * H), lambda i: (i, 0)),
            pl.BlockSpec((D, D), full),
            pl.BlockSpec((1, D), full),
            pl.BlockSpec((3 * D, 1), full),
            pl.BlockSpec((1, 1), full),
        ],
        out_specs=pl.BlockSpec((blk, 1), lambda i: (i, 0)),
        out_shape=jax.ShapeDtypeStruct((e, 1), jnp.float32),
    )(u, w1, b1, wcat, cst)


def kernel(s1, s2, p1, p2, x, w, edge_index, W1, b1, WL, bL, Wh1, bh1, Wh2, bh2):
    tsrc32, tdst32 = _build_tables(s1, s2, p1, p2, x, w)
    src = edge_index[0].astype(jnp.int32)
    dst = edge_index[1].astype(jnp.int32)
    cst = (bL + bh1 + bh2).reshape(1, 1).astype(jnp.float32)
    bf = jnp.bfloat16
    wcat = jnp.concatenate([WL, Wh1, Wh2], axis=0).astype(bf)
    u32 = _sc_gather_combine(tsrc32, tdst32, src, dst)
    return _tc_mlp(u32, W1.astype(bf), b1.reshape(1, D), wcat, cst)


# speedup vs baseline: 5.0108x; 1.0024x over previous
import dataclasses
import functools

import jax
import jax.numpy as jnp
from jax import lax
from jax.experimental import pallas as pl
from jax.experimental.pallas import tpu as pltpu
from jax.experimental.pallas import tpu_sc as plsc

D = 128
H = D // 2
NC = 2
NS = 16
NW = NC * NS


def _pack(feat):
    hi = jax.lax.bitcast_convert_type(feat[:, 0:H], jnp.uint16)
    lo = jax.lax.bitcast_convert_type(feat[:, H:2 * H], jnp.uint16)
    word = (hi.astype(jnp.uint32) << 16) | lo.astype(jnp.uint32)
    return jax.lax.bitcast_convert_type(word, jnp.int32)


def _prep_body(s1, s2, p1, p2, x, w, tsrc, tdst):
    xb = _pack(x[...].astype(jnp.bfloat16))
    wb = _pack(w[...].astype(jnp.bfloat16))
    zero = jnp.zeros_like(xb)
    tsrc[:, 0:H] = _pack((s1[...] + s2[...]).astype(jnp.bfloat16))
    tsrc[:, H:2 * H] = xb
    tsrc[:, 2 * H:3 * H] = wb
    tsrc[:, 3 * H:4 * H] = zero
    tdst[:, 0:H] = _pack((p1[...] + p2[...]).astype(jnp.bfloat16))
    tdst[:, H:2 * H] = xb
    tdst[:, 2 * H:3 * H] = wb
    tdst[:, 3 * H:4 * H] = zero


def _build_tables(s1, s2, p1, p2, x, w):
    n = s1.shape[0]
    blk = 2000
    grid = (n // blk,)
    in_spec = pl.BlockSpec((blk, D), lambda i: (i, 0))
    out_spec = pl.BlockSpec((blk, 4 * H), lambda i: (i, 0))
    return pl.pallas_call(
        _prep_body,
        grid=grid,
        in_specs=[in_spec] * 6,
        out_specs=[out_spec, out_spec],
        out_shape=[jax.ShapeDtypeStruct((n, 4 * H), jnp.int32)] * 2,
    )(s1, s2, p1, p2, x, w)


def _sc_gather_combine(tsrc32, tdst32, src_idx, dst_idx):
    e = src_idx.shape[0]
    dtin = tsrc32.shape[1]
    dt = 3 * H
    ew = e // NW
    wnd = 40
    nch = ew // wnd
    mesh = plsc.VectorSubcoreMesh(core_axis_name="c", subcore_axis_name="s")
    cp = pltpu.CompilerParams()
    if "needs_layout_passes" in pltpu.CompilerParams.__dataclass_fields__:
        cp = dataclasses.replace(cp, needs_layout_passes=False)

    @functools.partial(
        pl.kernel,
        out_type=jax.ShapeDtypeStruct((e, dt), jnp.int32),
        mesh=mesh,
        compiler_params=cp,
        scratch_types=[
            pltpu.VMEM((ew,), jnp.int32),
            pltpu.VMEM((ew,), jnp.int32),
            pltpu.VMEM((wnd, dtin), jnp.int32),
            pltpu.VMEM((wnd, dtin), jnp.int32),
            pltpu.VMEM((wnd, dtin), jnp.int32),
            pltpu.VMEM((wnd, dtin), jnp.int32),
            pltpu.VMEM((wnd, dt), jnp.int32),
            pltpu.VMEM((wnd, dt), jnp.int32),
            pltpu.SemaphoreType.DMA,
            pltpu.SemaphoreType.DMA,
            pltpu.SemaphoreType.DMA,
            pltpu.SemaphoreType.DMA,
            pltpu.SemaphoreType.DMA,
            pltpu.SemaphoreType.DMA,
        ],
    )
    def k(tsrc_hbm, tdst_hbm, si_hbm, di_hbm, u_hbm,
          si_v, di_v, a0_v, a1_v, b0_v, b1_v, u0_v, u1_v,
          sa0, sa1, sb0, sb1, su0, su1):
        wid = lax.axis_index("s") * NC + lax.axis_index("c")
        base = wid * ew
        pltpu.sync_copy(si_hbm.at[pl.ds(base, ew)], si_v)
        pltpu.sync_copy(di_hbm.at[pl.ds(base, ew)], di_v)

        a_bufs, b_bufs, u_bufs = (a0_v, a1_v), (b0_v, b1_v), (u0_v, u1_v)
        sa, sb, su = (sa0, sa1), (sb0, sb1), (su0, su1)

        def fire(g, pb):
            off = g * wnd
            pltpu.async_copy(tsrc_hbm.at[si_v.at[pl.ds(off, wnd)]],
                             a_bufs[pb], sa[pb])
            pltpu.async_copy(tdst_hbm.at[di_v.at[pl.ds(off, wnd)]],
                             b_bufs[pb], sb[pb])

        fire(0, 0)
        fire(1, 1)

        @pl.loop(0, nch, step=2)
        def _chunk(i):
            for pb in range(2):
                g = i + pb
                pltpu.make_async_copy(tsrc_hbm.at[pl.ds(0, wnd)],
                                      a_bufs[pb], sa[pb]).wait()
                pltpu.make_async_copy(tsrc_hbm.at[pl.ds(0, wnd)],
                                      b_bufs[pb], sb[pb]).wait()

                @pl.when(i >= 2)
                def _():
                    pltpu.make_async_copy(u_bufs[pb],
                                          u_hbm.at[pl.ds(base, wnd)],
                                          su[pb]).wait()

                a_v, b_v, u_v = a_bufs[pb], b_bufs[pb], u_bufs[pb]

                @pl.loop(0, wnd)
                def _row(r):
                    for j in range(dt // 16):
                        slc = (r, pl.ds(j * 16, 16))
                        av = plsc.bitcast(a_v.at[*slc][...], jnp.bfloat16)
                        bv = plsc.bitcast(b_v.at[*slc][...], jnp.bfloat16)
                        res = av + bv if j < 4 else av * bv
                        u_v.at[*slc][...] = plsc.bitcast(res, jnp.int32)

                pltpu.async_copy(u_v, u_hbm.at[pl.ds(base + g * wnd, wnd)],
                                 su[pb])

                @pl.when(g + 2 < nch)
                def _():
                    fire(g + 2, pb)

        for pb in range(2):
            pltpu.make_async_copy(u_bufs[pb], u_hbm.at[pl.ds(base, wnd)],
                                  su[pb]).wait()

    return k(tsrc32, tdst32, src_idx, dst_idx)


def _leaky(v):
    return jnp.maximum(v, 0.01 * v)


def _unpack(sec):
    hi = jax.lax.bitcast_convert_type(sec & jnp.uint32(0xFFFF0000),
                                      jnp.float32)
    lo = jax.lax.bitcast_convert_type(sec << 16, jnp.float32)
    return jnp.concatenate([hi, lo], axis=1)


def _mlp_body(u, w1, b1, wcat, cst, o):
    uu = jax.lax.bitcast_convert_type(u[...], jnp.uint32)
    c = _unpack(uu[:, 0:H])
    xp = _unpack(uu[:, H:2 * H])
    wp = _unpack(uu[:, 2 * H:3 * H])
    z = jnp.dot(_leaky(c).astype(jnp.bfloat16), w1[...],
                preferred_element_type=jnp.float32) + b1[...]
    a2 = jnp.concatenate([_leaky(z), xp, wp], axis=1).astype(jnp.bfloat16)
    blk = u.shape[0]
    i = pl.program_id(0)
    o[pl.ds(i * blk, blk)] = (jnp.dot(a2, wcat[...],
                                      preferred_element_type=jnp.float32)
                              + cst[...])[:, 0]


def _tc_mlp(u, w1, b1, wcat, cst):
    e = u.shape[0]
    blk = 6400
    grid = (e // blk,)
    full = lambda i: (0, 0)
    return pl.pallas_call(
        _mlp_body,
        grid=grid,
        in_specs=[
            pl.BlockSpec((blk, 3 * H), lambda i: (i, 0)),
            pl.BlockSpec((D, D), full),
            pl.BlockSpec((1, D), full),
            pl.BlockSpec((3 * D, 1), full),
            pl.BlockSpec((1, 1), full),
        ],
        out_specs=pl.BlockSpec((e,), lambda i: (0,)),
        out_shape=jax.ShapeDtypeStruct((e,), jnp.float32),
    )(u, w1, b1, wcat, cst)


def kernel(s1, s2, p1, p2, x, w, edge_index, W1, b1, WL, bL, Wh1, bh1, Wh2, bh2):
    tsrc32, tdst32 = _build_tables(s1, s2, p1, p2, x, w)
    src = edge_index[0].astype(jnp.int32)
    dst = edge_index[1].astype(jnp.int32)
    cst = (bL + bh1 + bh2).reshape(1, 1).astype(jnp.float32)
    bf = jnp.bfloat16
    wcat = jnp.concatenate([WL, Wh1, Wh2], axis=0).astype(bf)
    u32 = _sc_gather_combine(tsrc32, tdst32, src, dst)
    out = _tc_mlp(u32, W1.astype(bf), b1.reshape(1, D), wcat, cst)
    return out.reshape(-1, 1)


# speedup vs baseline: 5.1454x; 1.0269x over previous
import dataclasses
import functools

import jax
import jax.numpy as jnp
from jax import lax
from jax.experimental import pallas as pl
from jax.experimental.pallas import tpu as pltpu
from jax.experimental.pallas import tpu_sc as plsc

D = 128
H = D // 2
NC = 2
NS = 16
NW = NC * NS


def _pack(feat):
    hi = jax.lax.bitcast_convert_type(feat[:, 0:H], jnp.uint16)
    lo = jax.lax.bitcast_convert_type(feat[:, H:2 * H], jnp.uint16)
    word = (hi.astype(jnp.uint32) << 16) | lo.astype(jnp.uint32)
    return jax.lax.bitcast_convert_type(word, jnp.int32)


def _prep_body(s1, s2, p1, p2, x, w, tsrc, tdst):
    xb = _pack(x[...].astype(jnp.bfloat16))
    wb = _pack(w[...].astype(jnp.bfloat16))
    zero = jnp.zeros_like(xb)
    tsrc[:, 0:H] = _pack((s1[...] + s2[...]).astype(jnp.bfloat16))
    tsrc[:, H:2 * H] = xb
    tsrc[:, 2 * H:3 * H] = wb
    tsrc[:, 3 * H:4 * H] = zero
    tdst[:, 0:H] = _pack((p1[...] + p2[...]).astype(jnp.bfloat16))
    tdst[:, H:2 * H] = xb
    tdst[:, 2 * H:3 * H] = wb
    tdst[:, 3 * H:4 * H] = zero


def _build_tables(s1, s2, p1, p2, x, w):
    n = s1.shape[0]
    blk = 2000
    grid = (n // blk,)
    in_spec = pl.BlockSpec((blk, D), lambda i: (i, 0))
    out_spec = pl.BlockSpec((blk, 4 * H), lambda i: (i, 0))
    return pl.pallas_call(
        _prep_body,
        grid=grid,
        in_specs=[in_spec] * 6,
        out_specs=[out_spec, out_spec],
        out_shape=[jax.ShapeDtypeStruct((n, 4 * H), jnp.int32)] * 2,
    )(s1, s2, p1, p2, x, w)


def _sc_gather_combine(tsrc32, tdst32, ei_flat):
    e = ei_flat.shape[0] // 2
    dtin = tsrc32.shape[1]
    dt = 3 * H
    ew = e // NW
    wnd = 40
    nch = ew // wnd
    mesh = plsc.VectorSubcoreMesh(core_axis_name="c", subcore_axis_name="s")
    cp = pltpu.CompilerParams()
    if "needs_layout_passes" in pltpu.CompilerParams.__dataclass_fields__:
        cp = dataclasses.replace(cp, needs_layout_passes=False)

    nbuf = 4
    scr = [pltpu.VMEM((ew,), jnp.int32), pltpu.VMEM((ew,), jnp.int32)]
    scr += [pltpu.VMEM((wnd, dtin), jnp.int32)] * (2 * nbuf)
    scr += [pltpu.VMEM((wnd, dt), jnp.int32)] * 2
    scr += [pltpu.SemaphoreType.DMA] * (2 * nbuf + 2)

    @functools.partial(
        pl.kernel,
        out_type=jax.ShapeDtypeStruct((e, dt), jnp.int32),
        mesh=mesh,
        compiler_params=cp,
        scratch_types=scr,
    )
    def k(tsrc_hbm, tdst_hbm, ei_hbm, u_hbm, si_v, di_v, *rest):
        a_bufs = rest[0:nbuf]
        b_bufs = rest[nbuf:2 * nbuf]
        u_bufs = rest[2 * nbuf:2 * nbuf + 2]
        sa = rest[2 * nbuf + 2:3 * nbuf + 2]
        sb = rest[3 * nbuf + 2:4 * nbuf + 2]
        su = rest[4 * nbuf + 2:4 * nbuf + 4]
        wid = lax.axis_index("s") * NC + lax.axis_index("c")
        base = wid * ew
        pltpu.sync_copy(ei_hbm.at[pl.ds(base, ew)], si_v)
        pltpu.sync_copy(ei_hbm.at[pl.ds(e + base, ew)], di_v)

        def fire(g, pb):
            off = g * wnd
            pltpu.async_copy(tsrc_hbm.at[si_v.at[pl.ds(off, wnd)]],
                             a_bufs[pb], sa[pb])
            pltpu.async_copy(tdst_hbm.at[di_v.at[pl.ds(off, wnd)]],
                             b_bufs[pb], sb[pb])

        for pb in range(nbuf):
            fire(pb, pb)

        def do_chunk(g, pb, ub, tail=False):
            pltpu.make_async_copy(tsrc_hbm.at[pl.ds(0, wnd)],
                                  a_bufs[pb], sa[pb]).wait()
            pltpu.make_async_copy(tsrc_hbm.at[pl.ds(0, wnd)],
                                  b_bufs[pb], sb[pb]).wait()

            def drain_u():
                pltpu.make_async_copy(u_bufs[ub],
                                      u_hbm.at[pl.ds(base, wnd)],
                                      su[ub]).wait()

            if tail:
                drain_u()
            else:
                pl.when(g >= 2)(drain_u)

            a_v, b_v, u_v = a_bufs[pb], b_bufs[pb], u_bufs[ub]

            @pl.loop(0, wnd)
            def _row(r):
                for j in range(dt // 16):
                    slc = (r, pl.ds(j * 16, 16))
                    av = plsc.bitcast(a_v.at[*slc][...], jnp.bfloat16)
                    bv = plsc.bitcast(b_v.at[*slc][...], jnp.bfloat16)
                    res = av + bv if j < 4 else av * bv
                    u_v.at[*slc][...] = plsc.bitcast(res, jnp.int32)

            pltpu.async_copy(u_v, u_hbm.at[pl.ds(base + g * wnd, wnd)],
                             su[ub])

            if not tail:
                pl.when(g + nbuf < nch)(lambda: fire(g + nbuf, pb))

        body = nch - (nch % nbuf)

        @pl.loop(0, body, step=nbuf)
        def _chunk(i):
            for pb in range(nbuf):
                do_chunk(i + pb, pb, pb % 2)

        for g in range(body, nch):
            do_chunk(g, g % nbuf, g % 2, tail=True)

        for ub in range(2):
            pltpu.make_async_copy(u_bufs[ub], u_hbm.at[pl.ds(base, wnd)],
                                  su[ub]).wait()

    return k(tsrc32, tdst32, ei_flat)


def _leaky(v):
    return jnp.maximum(v, 0.01 * v)


def _unpack(sec):
    hi = jax.lax.bitcast_convert_type(sec & jnp.uint32(0xFFFF0000),
                                      jnp.float32)
    lo = jax.lax.bitcast_convert_type(sec << 16, jnp.float32)
    return jnp.concatenate([hi, lo], axis=1)


def _mlp_body(u, w1, b1, wcat, cst, o):
    uu = jax.lax.bitcast_convert_type(u[...], jnp.uint32)
    c = _unpack(uu[:, 0:H])
    xp = _unpack(uu[:, H:2 * H])
    wp = _unpack(uu[:, 2 * H:3 * H])
    z = jnp.dot(_leaky(c).astype(jnp.bfloat16), w1[...],
                preferred_element_type=jnp.float32) + b1[...]
    a2 = jnp.concatenate([_leaky(z), xp, wp], axis=1).astype(jnp.bfloat16)
    blk = u.shape[0]
    i = pl.program_id(0)
    o[pl.ds(i * blk, blk)] = (jnp.dot(a2, wcat[...],
                                      preferred_element_type=jnp.float32)
                              + cst[...])[:, 0]


def _tc_mlp(u, w1, b1, wcat, cst):
    e = u.shape[0]
    blk = 6400
    grid = (e // blk,)
    full = lambda i: (0, 0)
    return pl.pallas_call(
        _mlp_body,
        grid=grid,
        in_specs=[
            pl.BlockSpec((blk, 3 * H), lambda i: (i, 0)),
            pl.BlockSpec((D, D), full),
            pl.BlockSpec((1, D), full),
            pl.BlockSpec((3 * D, 1), full),
            pl.BlockSpec((1, 1), full),
        ],
        out_specs=pl.BlockSpec((e,), lambda i: (0,)),
        out_shape=jax.ShapeDtypeStruct((e,), jnp.float32),
    )(u, w1, b1, wcat, cst)


def kernel(s1, s2, p1, p2, x, w, edge_index, W1, b1, WL, bL, Wh1, bh1, Wh2, bh2):
    tsrc32, tdst32 = _build_tables(s1, s2, p1, p2, x, w)
    cst = (bL + bh1 + bh2).reshape(1, 1).astype(jnp.float32)
    bf = jnp.bfloat16
    wcat = jnp.concatenate([WL, Wh1, Wh2], axis=0).astype(bf)
    u32 = _sc_gather_combine(tsrc32, tdst32,
                             edge_index.astype(jnp.int32).reshape(-1))
    out = _tc_mlp(u32, W1.astype(bf), b1.reshape(1, D), wcat, cst)
    return out.reshape(-1, 1)


# speedup vs baseline: 5.2120x; 1.0129x over previous
import dataclasses
import functools

import jax
import jax.numpy as jnp
from jax import lax
from jax.experimental import pallas as pl
from jax.experimental.pallas import tpu as pltpu
from jax.experimental.pallas import tpu_sc as plsc

D = 128
H = D // 2
NC = 2
NS = 16
NW = NC * NS


def _pack(feat):
    hi = jax.lax.bitcast_convert_type(feat[:, 0:H], jnp.uint16)
    lo = jax.lax.bitcast_convert_type(feat[:, H:2 * H], jnp.uint16)
    word = (hi.astype(jnp.uint32) << 16) | lo.astype(jnp.uint32)
    return jax.lax.bitcast_convert_type(word, jnp.int32)


def _prep_body(s1, s2, p1, p2, x, w, tsrc, tdst):
    xb = _pack(x[...].astype(jnp.bfloat16))
    wb = _pack(w[...].astype(jnp.bfloat16))
    zero = jnp.zeros_like(xb)
    tsrc[:, 0:H] = _pack((s1[...] + s2[...]).astype(jnp.bfloat16))
    tsrc[:, H:2 * H] = xb
    tsrc[:, 2 * H:3 * H] = wb
    tsrc[:, 3 * H:4 * H] = zero
    tdst[:, 0:H] = _pack((p1[...] + p2[...]).astype(jnp.bfloat16))
    tdst[:, H:2 * H] = xb
    tdst[:, 2 * H:3 * H] = wb
    tdst[:, 3 * H:4 * H] = zero


def _build_tables(s1, s2, p1, p2, x, w):
    n = s1.shape[0]
    blk = 2000
    grid = (n // blk,)
    in_spec = pl.BlockSpec((blk, D), lambda i: (i, 0))
    out_spec = pl.BlockSpec((blk, 4 * H), lambda i: (i, 0))
    return pl.pallas_call(
        _prep_body,
        grid=grid,
        in_specs=[in_spec] * 6,
        out_specs=[out_spec, out_spec],
        out_shape=[jax.ShapeDtypeStruct((n, 4 * H), jnp.int32)] * 2,
    )(s1, s2, p1, p2, x, w)


def _sc_gather_combine(tsrc32, tdst32, ei_flat):
    e = ei_flat.shape[0] // 2
    dtin = tsrc32.shape[1]
    dt = 3 * H
    ew = e // NW
    wnd = 40
    nch = ew // wnd
    mesh = plsc.VectorSubcoreMesh(core_axis_name="c", subcore_axis_name="s")
    cp = pltpu.CompilerParams()
    if "needs_layout_passes" in pltpu.CompilerParams.__dataclass_fields__:
        cp = dataclasses.replace(cp, needs_layout_passes=False)

    nbuf = 4
    scr = [pltpu.VMEM((ew,), jnp.int32), pltpu.VMEM((ew,), jnp.int32)]
    scr += [pltpu.VMEM((wnd, dtin), jnp.int32)] * (2 * nbuf)
    scr += [pltpu.VMEM((wnd, dt), jnp.int32)] * 2
    scr += [pltpu.SemaphoreType.DMA] * (2 * nbuf + 2)

    @functools.partial(
        pl.kernel,
        out_type=jax.ShapeDtypeStruct((e, dt), jnp.int32),
        mesh=mesh,
        compiler_params=cp,
        scratch_types=scr,
    )
    def k(tsrc_hbm, tdst_hbm, ei_hbm, u_hbm, si_v, di_v, *rest):
        a_bufs = rest[0:nbuf]
        b_bufs = rest[nbuf:2 * nbuf]
        u_bufs = rest[2 * nbuf:2 * nbuf + 2]
        sa = rest[2 * nbuf + 2:3 * nbuf + 2]
        sb = rest[3 * nbuf + 2:4 * nbuf + 2]
        su = rest[4 * nbuf + 2:4 * nbuf + 4]
        wid = lax.axis_index("s") * NC + lax.axis_index("c")
        base = wid * ew
        pltpu.sync_copy(ei_hbm.at[pl.ds(base, ew)], si_v)
        pltpu.sync_copy(ei_hbm.at[pl.ds(e + base, ew)], di_v)

        def fire(g, pb):
            off = g * wnd
            pltpu.async_copy(tsrc_hbm.at[si_v.at[pl.ds(off, wnd)]],
                             a_bufs[pb], sa[pb])
            pltpu.async_copy(tdst_hbm.at[di_v.at[pl.ds(off, wnd)]],
                             b_bufs[pb], sb[pb])

        for pb in range(nbuf):
            fire(pb, pb)

        def do_chunk(g, pb, ub, tail=False):
            pltpu.make_async_copy(tsrc_hbm.at[pl.ds(0, wnd)],
                                  a_bufs[pb], sa[pb]).wait()
            pltpu.make_async_copy(tsrc_hbm.at[pl.ds(0, wnd)],
                                  b_bufs[pb], sb[pb]).wait()

            def drain_u():
                pltpu.make_async_copy(u_bufs[ub],
                                      u_hbm.at[pl.ds(base, wnd)],
                                      su[ub]).wait()

            if tail:
                drain_u()
            else:
                pl.when(g >= 2)(drain_u)

            a_v, b_v, u_v = a_bufs[pb], b_bufs[pb], u_bufs[ub]

            @pl.loop(0, wnd)
            def _row(r):
                for j in range(dt // 16):
                    slc = (r, pl.ds(j * 16, 16))
                    av = plsc.bitcast(a_v.at[*slc][...], jnp.bfloat16)
                    bv = plsc.bitcast(b_v.at[*slc][...], jnp.bfloat16)
                    if j < 4:
                        cv = av + bv
                        res = jnp.maximum(cv, jnp.bfloat16(0.01) * cv)
                    else:
                        res = av * bv
                    u_v.at[*slc][...] = plsc.bitcast(res, jnp.int32)

            pltpu.async_copy(u_v, u_hbm.at[pl.ds(base + g * wnd, wnd)],
                             su[ub])

            if not tail:
                pl.when(g + nbuf < nch)(lambda: fire(g + nbuf, pb))

        body = nch - (nch % nbuf)

        @pl.loop(0, body, step=nbuf)
        def _chunk(i):
            for pb in range(nbuf):
                do_chunk(i + pb, pb, pb % 2)

        for g in range(body, nch):
            do_chunk(g, g % nbuf, g % 2, tail=True)

        for ub in range(2):
            pltpu.make_async_copy(u_bufs[ub], u_hbm.at[pl.ds(base, wnd)],
                                  su[ub]).wait()

    return k(tsrc32, tdst32, ei_flat)


def _leaky(v):
    return jnp.maximum(v, 0.01 * v)


def _unpack(sec):
    hi = jax.lax.bitcast_convert_type(sec & jnp.uint32(0xFFFF0000),
                                      jnp.float32)
    lo = jax.lax.bitcast_convert_type(sec << 16, jnp.float32)
    return jnp.concatenate([hi, lo], axis=1)


def _mlp_body(u, w1, b1, wcat, cst, o):
    uu = jax.lax.bitcast_convert_type(u[...], jnp.uint32)
    c = _unpack(uu[:, 0:H])
    xp = _unpack(uu[:, H:2 * H])
    wp = _unpack(uu[:, 2 * H:3 * H])
    z = jnp.dot(c.astype(jnp.bfloat16), w1[...],
                preferred_element_type=jnp.float32) + b1[...]
    a2 = jnp.concatenate([_leaky(z), xp, wp], axis=1).astype(jnp.bfloat16)
    blk = u.shape[0]
    i = pl.program_id(0)
    o[pl.ds(i * blk, blk)] = (jnp.dot(a2, wcat[...],
                                      preferred_element_type=jnp.float32)
                              + cst[...])[:, 0]


def _tc_mlp(u, w1, b1, wcat, cst):
    e = u.shape[0]
    blk = 6400
    grid = (e // blk,)
    full = lambda i: (0, 0)
    return pl.pallas_call(
        _mlp_body,
        grid=grid,
        in_specs=[
            pl.BlockSpec((blk, 3 * H), lambda i: (i, 0)),
            pl.BlockSpec((D, D), full),
            pl.BlockSpec((1, D), full),
            pl.BlockSpec((3 * D, 1), full),
            pl.BlockSpec((1, 1), full),
        ],
        out_specs=pl.BlockSpec((e,), lambda i: (0,)),
        out_shape=jax.ShapeDtypeStruct((e,), jnp.float32),
    )(u, w1, b1, wcat, cst)


def kernel(s1, s2, p1, p2, x, w, edge_index, W1, b1, WL, bL, Wh1, bh1, Wh2, bh2):
    tsrc32, tdst32 = _build_tables(s1, s2, p1, p2, x, w)
    cst = (bL + bh1 + bh2).reshape(1, 1).astype(jnp.float32)
    bf = jnp.bfloat16
    wcat = jnp.concatenate([WL, Wh1, Wh2], axis=0).astype(bf)
    u32 = _sc_gather_combine(tsrc32, tdst32,
                             edge_index.astype(jnp.int32).reshape(-1))
    out = _tc_mlp(u32, W1.astype(bf), b1.reshape(1, D), wcat, cst)
    return out.reshape(-1, 1)
